# baseline (speedup 1.0000x reference)
import functools

import jax
import jax.numpy as jnp
from jax import lax
from jax.experimental import pallas as pl
from jax.experimental.pallas import tpu as pltpu
from jax.experimental.pallas import tpu_sc as plsc

_INDICES = 1000000
_SIZE = 32
_BATCH = 16384
_PACK = 128 // _SIZE
_CH = 128


def _build():
    info = plsc.get_sparse_core_info()
    num_cores, num_subcores = info.num_cores, info.num_subcores
    num_workers = num_cores * num_subcores
    b_per_w = _BATCH // num_workers
    lanes = info.num_lanes
    n_chunks = b_per_w // _CH
    mesh = plsc.VectorSubcoreMesh(core_axis_name="c", subcore_axis_name="s")

    @functools.partial(
        pl.kernel,
        mesh=mesh,
        out_type=jax.ShapeDtypeStruct((_BATCH, _SIZE), jnp.float32),
        scratch_types=[
            pltpu.VMEM((b_per_w,), jnp.int32),
            pltpu.VMEM((b_per_w,), jnp.int32),
            pltpu.VMEM((_CH, 128), jnp.float32),
            pltpu.VMEM((_CH, 128), jnp.float32),
            pltpu.VMEM((b_per_w, _SIZE), jnp.float32),
            pltpu.SemaphoreType.DMA,
        ],
        compiler_params=pltpu.CompilerParams(needs_layout_passes=False),
    )
    def gather_kernel(
        table_hbm, idx_hbm, out_hbm, idx_v, line_v, buf_a, buf_b, out_v, sem
    ):
        wid = lax.axis_index("s") * num_cores + lax.axis_index("c")
        base = wid * b_per_w
        pltpu.sync_copy(idx_hbm.at[pl.ds(base, b_per_w)], idx_v)

        def line_body(i, carry):
            r = idx_v[pl.ds(i * lanes, lanes)]
            line_v[pl.ds(i * lanes, lanes)] = r >> 2
            return carry

        lax.fori_loop(0, b_per_w // lanes, line_body, 0)

        bufs = [buf_a, buf_b]
        lane_iota = lax.iota(jnp.int32, lanes)

        def start(ci):
            return pltpu.async_copy(
                table_hbm.at[line_v.at[pl.ds(ci * _CH, _CH)]],
                bufs[ci % 2],
                sem,
            )

        def extract(ci):
            buf = bufs[ci % 2]

            def body(i, carry):
                r = idx_v[pl.ds(ci * _CH + i * lanes, lanes)]
                col0 = (r & 3) << 5
                kloc = lane_iota + i * lanes
                kglob = kloc + ci * _CH
                for c in range(_SIZE):
                    cvec = jnp.full((lanes,), c, jnp.int32)
                    vals = plsc.load_gather(buf, [kloc, col0 + cvec])
                    plsc.store_scatter(out_v, [kglob, cvec], vals)
                return carry

            lax.fori_loop(0, _CH // lanes, body, 0)

        copies = [start(0)]
        for ci in range(n_chunks):
            copies[ci].wait()
            if ci + 1 < n_chunks:
                copies.append(start(ci + 1))
            extract(ci)

        pltpu.sync_copy(out_v, out_hbm.at[pl.ds(base, b_per_w)])

    return gather_kernel


_gather = _build()


def kernel(x, update, weights):
    del update
    table4 = weights.reshape(_INDICES // _PACK, 128)
    return _gather(table4, x.astype(jnp.int32))


# speedup vs baseline: 1.7117x; 1.7117x over previous
import functools

import jax
import jax.numpy as jnp
from jax import lax
from jax.experimental import pallas as pl
from jax.experimental.pallas import tpu as pltpu
from jax.experimental.pallas import tpu_sc as plsc

_INDICES = 1000000
_SIZE = 32
_BATCH = 16384


def _build():
    info = plsc.get_sparse_core_info()
    num_cores, num_subcores = info.num_cores, info.num_subcores
    num_workers = num_cores * num_subcores
    b_per_w = _BATCH // num_workers
    lanes = info.num_lanes
    mesh = plsc.VectorSubcoreMesh(core_axis_name="c", subcore_axis_name="s")

    @functools.partial(
        pl.kernel,
        mesh=mesh,
        out_type=jax.ShapeDtypeStruct((_BATCH, _SIZE), jnp.float32),
        scratch_types=[
            pltpu.VMEM((b_per_w,), jnp.int32),
            pltpu.VMEM((b_per_w, _SIZE), jnp.float32),
            pltpu.SemaphoreType.DMA,
        ],
    )
    def gather_kernel(table_hbm, idx_hbm, out_hbm, idx_v, rows_v, sem):
        wid = lax.axis_index("s") * num_cores + lax.axis_index("c")
        base = wid * b_per_w
        pltpu.sync_copy(idx_hbm.at[pl.ds(base, b_per_w)], idx_v)

        def body(i, carry):
            vec = idx_v[pl.ds(i * lanes, lanes)]
            for j in range(lanes):
                r = vec[j]
                k = i * lanes + j
                pltpu.async_copy(table_hbm.at[r], rows_v.at[k], sem)
            return carry

        lax.fori_loop(0, b_per_w // lanes, body, 0)
        pltpu.make_async_copy(
            table_hbm.at[pl.ds(0, b_per_w)], rows_v, sem
        ).wait()
        pltpu.sync_copy(rows_v, out_hbm.at[pl.ds(base, b_per_w)])

    return gather_kernel


_gather = _build()


def kernel(x, update, weights):
    del update
    return _gather(weights * jnp.float32(1.0), x.astype(jnp.int32))


# speedup vs baseline: 2.8318x; 1.6544x over previous
import functools

import jax
import jax.numpy as jnp
from jax import lax
from jax.experimental import pallas as pl
from jax.experimental.pallas import tpu as pltpu
from jax.experimental.pallas import tpu_sc as plsc

_INDICES = 1000000
_SIZE = 32
_BATCH = 16384


def _build():
    info = plsc.get_sparse_core_info()
    num_cores, num_subcores = info.num_cores, info.num_subcores
    num_workers = num_cores * num_subcores
    b_per_w = _BATCH // num_workers
    lanes = info.num_lanes
    mesh = plsc.VectorSubcoreMesh(core_axis_name="c", subcore_axis_name="s")

    @functools.partial(
        pl.kernel,
        mesh=mesh,
        out_type=jax.ShapeDtypeStruct((_BATCH, _SIZE), jnp.float32),
        scratch_types=[
            pltpu.VMEM((b_per_w,), jnp.int32),
            pltpu.VMEM((b_per_w, _SIZE), jnp.float32),
            pltpu.SemaphoreType.DMA,
        ],
    )
    def gather_kernel(table_hbm, idx_hbm, out_hbm, idx_v, rows_v, sem):
        wid = lax.axis_index("s") * num_cores + lax.axis_index("c")
        base = wid * b_per_w
        pltpu.sync_copy(idx_hbm.at[pl.ds(base, b_per_w)], idx_v)

        def body(i, carry):
            vec = idx_v[pl.ds(i * lanes, lanes)]
            for j in range(lanes):
                r = vec[j]
                k = i * lanes + j
                pltpu.async_copy(table_hbm.at[0, r], rows_v.at[k], sem)
            return carry

        lax.fori_loop(0, b_per_w // lanes, body, 0)
        pltpu.make_async_copy(
            table_hbm.at[0, pl.ds(0, b_per_w)], rows_v, sem
        ).wait()
        pltpu.sync_copy(rows_v, out_hbm.at[pl.ds(base, b_per_w)])

    return gather_kernel


_gather = _build()


def kernel(x, update, weights):
    del update
    return _gather(weights.reshape(1, _INDICES, _SIZE), x.astype(jnp.int32))
